# baseline (speedup 1.0000x reference)
import functools
import jax
import jax.numpy as jnp
from jax import lax
from jax.experimental import pallas as pl
from jax.experimental.pallas import tpu as pltpu
from jax.experimental.pallas import tpu_sc as plsc

_E = 8
_K = 2
_TT = 256
_R = 128
_NC = 2
_NS = 16


def _tc1_body(rw_ref, x_ref, sg_ref, su_ref, sd_ref,
              idx_ref, p_ref, shared_ref):
    xb = x_ref[...]
    logits = jnp.dot(xb, rw_ref[...], preferred_element_type=jnp.float32)
    m = jnp.max(logits, axis=-1, keepdims=True)
    ex = jnp.exp(logits - m)
    p = ex / jnp.sum(ex, axis=-1, keepdims=True)
    lane = jax.lax.broadcasted_iota(jnp.int32, p.shape, 1)
    m1 = jnp.max(p, axis=-1, keepdims=True)
    i1 = jnp.min(jnp.where(p == m1, lane, _E), axis=-1, keepdims=True)
    pm = jnp.where(lane == i1, -jnp.inf, p)
    m2 = jnp.max(pm, axis=-1, keepdims=True)
    i2 = jnp.min(jnp.where(pm == m2, lane, _E), axis=-1, keepdims=True)
    idx_ref[...] = jnp.where(lane == 0, i1, jnp.where(lane == 1, i2, 0))
    p_ref[...] = jnp.where(lane == 0, m1, jnp.where(lane == 1, m2, 0.0))
    g = jnp.dot(xb, sg_ref[...], preferred_element_type=jnp.float32)
    u = jnp.dot(xb, su_ref[...], preferred_element_type=jnp.float32)
    hmid = (g * jax.nn.sigmoid(g)) * u
    shared_ref[...] = jnp.dot(hmid, sd_ref[...],
                              preferred_element_type=jnp.float32)


def _tc2_body(etile_ref, nvalid_ref, x_ref, gate_ref, up_ref, down_ref,
              y_ref):
    i = pl.program_id(0)

    @pl.when(i < nvalid_ref[0])
    def _():
        xb = x_ref[...]
        g = jnp.dot(xb, gate_ref[0], preferred_element_type=jnp.float32)
        u = jnp.dot(xb, up_ref[0], preferred_element_type=jnp.float32)
        hmid = (g * jax.nn.sigmoid(g)) * u
        y_ref[...] = jnp.dot(hmid, down_ref[0],
                             preferred_element_type=jnp.float32)


def _build_sc_dispatch(T, h, NPAD, NT2_PAD):
    NW = _NC * _NS
    TPW = T // NW
    NCH = TPW // 16
    NCH_ALL = T // 16
    mesh = plsc.VectorSubcoreMesh(core_axis_name="c", subcore_axis_name="s")

    @functools.partial(
        pl.kernel, mesh=mesh,
        compiler_params=pltpu.CompilerParams(needs_layout_passes=False),
        out_type=[
            jax.ShapeDtypeStruct((NPAD, h), jnp.float32),
            jax.ShapeDtypeStruct((T,), jnp.int32),
            jax.ShapeDtypeStruct((T,), jnp.int32),
            jax.ShapeDtypeStruct((NT2_PAD,), jnp.int32),
            jax.ShapeDtypeStruct((16,), jnp.int32),
        ],
        scratch_types=[
            pltpu.VMEM((T,), jnp.int32),
            pltpu.VMEM((T,), jnp.int32),
            pltpu.VMEM((16, h), jnp.float32),
            pltpu.VMEM((16, h), jnp.float32),
            pltpu.VMEM((TPW,), jnp.int32),
            pltpu.VMEM((TPW,), jnp.int32),
            pltpu.VMEM((16,), jnp.int32),
            pltpu.VMEM((16,), jnp.int32),
            pltpu.VMEM((16,), jnp.int32),
            pltpu.VMEM((16,), jnp.int32),
            pltpu.VMEM((16,), jnp.int32),
            pltpu.VMEM((16,), jnp.int32),
            pltpu.VMEM((NT2_PAD,), jnp.int32),
            pltpu.VMEM((16,), jnp.int32),
            pltpu.SemaphoreType.DMA,
            pltpu.SemaphoreType.DMA,
            pltpu.SemaphoreType.DMA,
            pltpu.SemaphoreType.DMA,
            pltpu.SemaphoreType.DMA,
            pltpu.SemaphoreType.DMA,
        ],
    )
    def dispatch(x_hbm, i1_hbm, i2_hbm,
                 xd_hbm, pos0_hbm, pos1_hbm, et_hbm, nv_hbm,
                 i1_v, i2_v, xba, xbb, posbuf0, posbuf1,
                 idx0a, idx0b, idx1a, idx1b, run_ref, gend_ref,
                 etile_buf, nv_buf, sla, slb, s0a, s0b, s1a, s1b):
        cid = lax.axis_index("c")
        sid = lax.axis_index("s")
        wid = sid * _NC + cid
        base_tok = wid * TPW
        lanes = lax.broadcasted_iota(jnp.int32, (16,), 0)
        zero16 = jnp.zeros((16,), jnp.int32)

        pltpu.sync_copy(i1_hbm, i1_v)
        pltpu.sync_copy(i2_hbm, i2_v)

        snap_c = wid * NCH

        def count_half(iv, run0):
            def body(c, carry):
                run, snap = carry
                snap = jnp.where(c == snap_c, run, snap)
                ev = iv[pl.ds(c * 16, 16)]
                for e in range(_E):
                    m = ev == e
                    cnt = plsc.all_reduce_population_count(m)
                    run = run + jnp.where(lanes == e, cnt, 0)
                return run, snap
            return lax.fori_loop(0, NCH_ALL, body, (run0, zero16))

        run1, snap0 = count_half(i1_v, zero16)
        totals, snap1 = count_half(i2_v, run1)

        G = jnp.where(lanes < _E, ((totals + (_R - 1)) // _R) * _R, 0)
        gend = plsc.cumsum(G)
        start = gend - G
        gend_ref[...] = gend

        @pl.when(wid == 0)
        def _():
            for c3 in range(NT2_PAD // 16):
                tilestart = (lanes + c3 * 16) * _R
                acc = jnp.zeros((16,), jnp.int32)
                for e in range(_E):
                    ge = jnp.sum(jnp.where(lanes == e, gend, 0))
                    acc = acc + (tilestart >= ge).astype(jnp.int32)
                etile_buf[pl.ds(c3 * 16, 16)] = jnp.minimum(acc, _E - 1)
            pltpu.sync_copy(etile_buf, et_hbm)
            nv = jnp.sum(jnp.where(lanes == _E - 1, gend, 0)) // _R
            nv_buf[...] = jnp.broadcast_to(nv, (16,))
            pltpu.sync_copy(nv_buf, nv_hbm)

        def emit_positions(iv, snap, posbuf):
            run_ref[...] = start + snap
            for cc in range(NCH):
                ev = iv[pl.ds(base_tok + cc * 16, 16)]
                basev = plsc.load_gather(run_ref, [ev])
                rank = jnp.zeros((16,), jnp.int32)
                upd = jnp.zeros((16,), jnp.int32)
                for e in range(_E):
                    m = ev == e
                    r = plsc.cumsum(m.astype(jnp.int32)) - 1
                    rank = jnp.where(m, r, rank)
                    cnt = plsc.all_reduce_population_count(m)
                    upd = upd + jnp.where(lanes == e, cnt, 0)
                run_ref[...] = run_ref[...] + upd
                posbuf[pl.ds(cc * 16, 16)] = basev + rank

        emit_positions(i1_v, snap0, posbuf0)
        emit_positions(i2_v, snap1, posbuf1)
        pltpu.sync_copy(posbuf0, pos0_hbm.at[pl.ds(base_tok, TPW)])
        pltpu.sync_copy(posbuf1, pos1_hbm.at[pl.ds(base_tok, TPW)])

        xbuf = (xba, xbb)
        idx0 = (idx0a, idx0b)
        idx1 = (idx1a, idx1b)
        seml = (sla, slb)
        sem0 = (s0a, s0b)
        sem1 = (s1a, s1b)

        def load(cc, slot):
            return pltpu.async_copy(
                x_hbm.at[pl.ds(base_tok + cc * 16, 16)], xbuf[slot],
                seml[slot])

        def wait_scat(ps, slot):
            if ps[slot] is not None:
                ps[slot][0].wait()
                ps[slot][1].wait()
                ps[slot] = None

        pend_load = load(0, 0)
        pend_scat = [None, None]
        for cc in range(NCH):
            slot = cc % 2
            cur_load = pend_load
            if cc + 1 < NCH:
                wait_scat(pend_scat, 1 - slot)
                pend_load = load(cc + 1, 1 - slot)
            cur_load.wait()
            wait_scat(pend_scat, slot)
            idx0[slot][...] = posbuf0[pl.ds(cc * 16, 16)]
            idx1[slot][...] = posbuf1[pl.ds(cc * 16, 16)]
            cp0 = pltpu.async_copy(xbuf[slot], xd_hbm.at[idx0[slot]],
                                   sem0[slot])
            cp1 = pltpu.async_copy(xbuf[slot], xd_hbm.at[idx1[slot]],
                                   sem1[slot])
            pend_scat[slot] = (cp0, cp1)
        wait_scat(pend_scat, 0)
        wait_scat(pend_scat, 1)

    return dispatch


def _build_sc_combine(T, h, NPAD):
    NW = _NC * _NS
    TPW = T // NW
    NCH = TPW // 16
    mesh = plsc.VectorSubcoreMesh(core_axis_name="c", subcore_axis_name="s")

    @functools.partial(
        pl.kernel, mesh=mesh,
        compiler_params=pltpu.CompilerParams(needs_layout_passes=False),
        out_type=jax.ShapeDtypeStruct((T, h), jnp.float32),
        scratch_types=[
            pltpu.VMEM((2, 16), jnp.int32),
            pltpu.VMEM((2, 16), jnp.int32),
            pltpu.VMEM((2, 16), jnp.float32),
            pltpu.VMEM((2, 16), jnp.float32),
            pltpu.VMEM((16, h), jnp.float32),
            pltpu.VMEM((16, h), jnp.float32),
            pltpu.VMEM((16, h), jnp.float32),
            pltpu.VMEM((16, h), jnp.float32),
            pltpu.VMEM((16, h), jnp.float32),
            pltpu.VMEM((16, h), jnp.float32),
            pltpu.VMEM((16, h), jnp.float32),
            pltpu.SemaphoreType.DMA,
            pltpu.SemaphoreType.DMA,
            pltpu.SemaphoreType.DMA,
            pltpu.SemaphoreType.DMA,
            pltpu.SemaphoreType.DMA,
            pltpu.SemaphoreType.DMA,
        ],
    )
    def combine(y_hbm, sh_hbm, pos0_hbm, pos1_hbm, pw0_hbm, pw1_hbm,
                out_hbm,
                idx0, idx1, p0c, p1c, y0a, y0b, y1a, y1b, sba, sbb, obuf,
                s0a, s0b, s1a, s1b, ssa, ssb):
        cid = lax.axis_index("c")
        sid = lax.axis_index("s")
        wid = sid * _NC + cid
        base_tok = wid * TPW
        lanes = lax.broadcasted_iota(jnp.int32, (16,), 0)

        ybuf0 = (y0a, y0b)
        ybuf1 = (y1a, y1b)
        sbuf = (sba, sbb)
        sem0 = (s0a, s0b)
        sem1 = (s1a, s1b)
        sems = (ssa, ssb)

        def issue(cc, slot):
            base = base_tok + cc * 16
            pltpu.sync_copy(pos0_hbm.at[pl.ds(base, 16)], idx0.at[slot])
            pltpu.sync_copy(pos1_hbm.at[pl.ds(base, 16)], idx1.at[slot])
            pltpu.sync_copy(pw0_hbm.at[pl.ds(base, 16)], p0c.at[slot])
            pltpu.sync_copy(pw1_hbm.at[pl.ds(base, 16)], p1c.at[slot])
            cp0 = pltpu.async_copy(y_hbm.at[idx0.at[slot]], ybuf0[slot],
                                   sem0[slot])
            cp1 = pltpu.async_copy(y_hbm.at[idx1.at[slot]], ybuf1[slot],
                                   sem1[slot])
            cps = pltpu.async_copy(sh_hbm.at[pl.ds(base, 16)], sbuf[slot],
                                   sems[slot])
            return cp0, cp1, cps

        pend = issue(0, 0)
        for cc in range(NCH):
            slot = cc % 2
            cur = pend
            if cc + 1 < NCH:
                pend = issue(cc + 1, (cc + 1) % 2)
            cur[0].wait()
            cur[1].wait()
            cur[2].wait()

            p0v = p0c[slot]
            p1v = p1c[slot]
            yb0 = ybuf0[slot]
            yb1 = ybuf1[slot]
            sb = sbuf[slot]

            for r in range(16):
                w0 = jnp.sum(jnp.where(lanes == r, p0v, 0.0))
                w1 = jnp.sum(jnp.where(lanes == r, p1v, 0.0))

                def body(j, carry):
                    off = j * 64
                    for q in range(4):
                        o = off + q * 16
                        obuf[r, pl.ds(o, 16)] = (
                            w0 * yb0[r, pl.ds(o, 16)]
                            + w1 * yb1[r, pl.ds(o, 16)]
                            + sb[r, pl.ds(o, 16)])
                    return carry

                lax.fori_loop(0, h // 64, body, 0)
            pltpu.sync_copy(obuf, out_hbm.at[pl.ds(base_tok + cc * 16, 16)])

    return combine


def kernel(x, router_w, gate_w, up_w, down_w, shared_gate_w, shared_up_w,
           shared_down_w):
    b, s, h = x.shape
    d = gate_w.shape[2]
    T = b * s
    x_flat = x.reshape(T, h)
    NPAD = T * _K + _E * _R
    NT2 = NPAD // _R
    NT2_PAD = ((NT2 + 15) // 16) * 16

    nT = T // _TT
    idx8, p8, shared_out = pl.pallas_call(
        _tc1_body,
        grid=(nT,),
        in_specs=[
            pl.BlockSpec((h, _E), lambda j: (0, 0)),
            pl.BlockSpec((_TT, h), lambda j: (j, 0)),
            pl.BlockSpec((h, d), lambda j: (0, 0)),
            pl.BlockSpec((h, d), lambda j: (0, 0)),
            pl.BlockSpec((d, h), lambda j: (0, 0)),
        ],
        out_specs=[
            pl.BlockSpec((_TT, _E), lambda j: (j, 0)),
            pl.BlockSpec((_TT, _E), lambda j: (j, 0)),
            pl.BlockSpec((_TT, h), lambda j: (j, 0)),
        ],
        out_shape=[
            jax.ShapeDtypeStruct((T, _E), jnp.int32),
            jax.ShapeDtypeStruct((T, _E), jnp.float32),
            jax.ShapeDtypeStruct((T, h), jnp.float32),
        ],
    )(router_w, x_flat, shared_gate_w, shared_up_w, shared_down_w)

    i1 = idx8[:, 0]
    i2 = idx8[:, 1]
    p1 = p8[:, 0]
    p2 = p8[:, 1]

    dispatch = _build_sc_dispatch(T, h, NPAD, NT2_PAD)
    x_disp, pos0, pos1, e_of_tile, nv16 = dispatch(x_flat, i1, i2)

    grid_spec = pltpu.PrefetchScalarGridSpec(
        num_scalar_prefetch=2,
        grid=(NT2,),
        in_specs=[
            pl.BlockSpec((_R, h), lambda i, et, nv: (i, 0)),
            pl.BlockSpec((1, h, d), lambda i, et, nv: (et[i], 0, 0)),
            pl.BlockSpec((1, h, d), lambda i, et, nv: (et[i], 0, 0)),
            pl.BlockSpec((1, d, h), lambda i, et, nv: (et[i], 0, 0)),
        ],
        out_specs=pl.BlockSpec((_R, h), lambda i, et, nv: (i, 0)),
    )
    y = pl.pallas_call(
        _tc2_body,
        grid_spec=grid_spec,
        out_shape=jax.ShapeDtypeStruct((NPAD, h), jnp.float32),
    )(e_of_tile, nv16[:1], x_disp, gate_w, up_w, down_w)

    combine = _build_sc_combine(T, h, NPAD)
    out = combine(y, shared_out, pos0, pos1, p1, p2)
    return out.reshape(b, s, h)


# speedup vs baseline: 1.0083x; 1.0083x over previous
import functools
import jax
import jax.numpy as jnp
from jax import lax
from jax.experimental import pallas as pl
from jax.experimental.pallas import tpu as pltpu
from jax.experimental.pallas import tpu_sc as plsc

_E = 8
_K = 2
_TT = 256
_R = 128
_NC = 2
_NS = 16


def _tc1a_body(rw_ref, x_ref, idx_ref, p_ref):
    xb = x_ref[...]
    logits = jnp.dot(xb, rw_ref[...], preferred_element_type=jnp.float32)
    m = jnp.max(logits, axis=-1, keepdims=True)
    ex = jnp.exp(logits - m)
    p = ex / jnp.sum(ex, axis=-1, keepdims=True)
    lane = jax.lax.broadcasted_iota(jnp.int32, p.shape, 1)
    m1 = jnp.max(p, axis=-1, keepdims=True)
    i1 = jnp.min(jnp.where(p == m1, lane, _E), axis=-1, keepdims=True)
    pm = jnp.where(lane == i1, -jnp.inf, p)
    m2 = jnp.max(pm, axis=-1, keepdims=True)
    i2 = jnp.min(jnp.where(pm == m2, lane, _E), axis=-1, keepdims=True)
    idx_ref[...] = jnp.where(lane == 0, i1, jnp.where(lane == 1, i2, 0))
    p_ref[...] = jnp.where(lane == 0, m1, jnp.where(lane == 1, m2, 0.0))


def _tc1b_body(x_ref, sg_ref, su_ref, sd_ref, shared_ref):
    xb = x_ref[...]
    g = jnp.dot(xb, sg_ref[...], preferred_element_type=jnp.float32)
    u = jnp.dot(xb, su_ref[...], preferred_element_type=jnp.float32)
    hmid = (g * jax.nn.sigmoid(g)) * u
    shared_ref[...] = jnp.dot(hmid, sd_ref[...],
                              preferred_element_type=jnp.float32)


def _tc2_body(etile_ref, nvalid_ref, x_ref, gate_ref, up_ref, down_ref,
              y_ref):
    i = pl.program_id(0)

    @pl.when(i < nvalid_ref[0])
    def _():
        xb = x_ref[...]
        g = jnp.dot(xb, gate_ref[0], preferred_element_type=jnp.float32)
        u = jnp.dot(xb, up_ref[0], preferred_element_type=jnp.float32)
        hmid = (g * jax.nn.sigmoid(g)) * u
        y_ref[...] = jnp.dot(hmid, down_ref[0],
                             preferred_element_type=jnp.float32)


def _build_sc_dispatch(T, h, NPAD, NT2_PAD):
    NW = _NC * _NS
    TPW = T // NW
    NCH = TPW // 16
    NCH_ALL = T // 16
    mesh = plsc.VectorSubcoreMesh(core_axis_name="c", subcore_axis_name="s")

    @functools.partial(
        pl.kernel, mesh=mesh,
        compiler_params=pltpu.CompilerParams(needs_layout_passes=False),
        out_type=[
            jax.ShapeDtypeStruct((NPAD, h), jnp.float32),
            jax.ShapeDtypeStruct((T,), jnp.int32),
            jax.ShapeDtypeStruct((T,), jnp.int32),
            jax.ShapeDtypeStruct((NT2_PAD,), jnp.int32),
            jax.ShapeDtypeStruct((16,), jnp.int32),
        ],
        scratch_types=[
            pltpu.VMEM((T,), jnp.int32),
            pltpu.VMEM((T,), jnp.int32),
            pltpu.VMEM((16, h), jnp.float32),
            pltpu.VMEM((16, h), jnp.float32),
            pltpu.VMEM((TPW,), jnp.int32),
            pltpu.VMEM((TPW,), jnp.int32),
            pltpu.VMEM((16,), jnp.int32),
            pltpu.VMEM((16,), jnp.int32),
            pltpu.VMEM((16,), jnp.int32),
            pltpu.VMEM((16,), jnp.int32),
            pltpu.VMEM((16,), jnp.int32),
            pltpu.VMEM((16,), jnp.int32),
            pltpu.VMEM((NT2_PAD,), jnp.int32),
            pltpu.VMEM((16,), jnp.int32),
            pltpu.SemaphoreType.DMA,
            pltpu.SemaphoreType.DMA,
            pltpu.SemaphoreType.DMA,
            pltpu.SemaphoreType.DMA,
            pltpu.SemaphoreType.DMA,
            pltpu.SemaphoreType.DMA,
        ],
    )
    def dispatch(x_hbm, i1_hbm, i2_hbm,
                 xd_hbm, pos0_hbm, pos1_hbm, et_hbm, nv_hbm,
                 i1_v, i2_v, xba, xbb, posbuf0, posbuf1,
                 idx0a, idx0b, idx1a, idx1b, run_ref, gend_ref,
                 etile_buf, nv_buf, sla, slb, s0a, s0b, s1a, s1b):
        cid = lax.axis_index("c")
        sid = lax.axis_index("s")
        wid = sid * _NC + cid
        base_tok = wid * TPW
        lanes = lax.broadcasted_iota(jnp.int32, (16,), 0)
        zero16 = jnp.zeros((16,), jnp.int32)

        pltpu.sync_copy(i1_hbm, i1_v)
        pltpu.sync_copy(i2_hbm, i2_v)

        snap_c = wid * NCH

        def count_half(iv, run0):
            def body(c, carry):
                run, snap = carry
                snap = jnp.where(c == snap_c, run, snap)
                ev = iv[pl.ds(c * 16, 16)]
                for e in range(_E):
                    m = ev == e
                    cnt = plsc.all_reduce_population_count(m)
                    run = run + jnp.where(lanes == e, cnt, 0)
                return run, snap
            return lax.fori_loop(0, NCH_ALL, body, (run0, zero16))

        run1, snap0 = count_half(i1_v, zero16)
        totals, snap1 = count_half(i2_v, run1)

        G = jnp.where(lanes < _E, ((totals + (_R - 1)) // _R) * _R, 0)
        gend = plsc.cumsum(G)
        start = gend - G
        gend_ref[...] = gend

        @pl.when(wid == 0)
        def _():
            for c3 in range(NT2_PAD // 16):
                tilestart = (lanes + c3 * 16) * _R
                acc = jnp.zeros((16,), jnp.int32)
                for e in range(_E):
                    ge = jnp.sum(jnp.where(lanes == e, gend, 0))
                    acc = acc + (tilestart >= ge).astype(jnp.int32)
                etile_buf[pl.ds(c3 * 16, 16)] = jnp.minimum(acc, _E - 1)
            pltpu.sync_copy(etile_buf, et_hbm)
            nv = jnp.sum(jnp.where(lanes == _E - 1, gend, 0)) // _R
            nv_buf[...] = jnp.broadcast_to(nv, (16,))
            pltpu.sync_copy(nv_buf, nv_hbm)

        def emit_positions(iv, snap, posbuf):
            run_ref[...] = start + snap
            for cc in range(NCH):
                ev = iv[pl.ds(base_tok + cc * 16, 16)]
                basev = plsc.load_gather(run_ref, [ev])
                rank = jnp.zeros((16,), jnp.int32)
                upd = jnp.zeros((16,), jnp.int32)
                for e in range(_E):
                    m = ev == e
                    r = plsc.cumsum(m.astype(jnp.int32)) - 1
                    rank = jnp.where(m, r, rank)
                    cnt = plsc.all_reduce_population_count(m)
                    upd = upd + jnp.where(lanes == e, cnt, 0)
                run_ref[...] = run_ref[...] + upd
                posbuf[pl.ds(cc * 16, 16)] = basev + rank

        emit_positions(i1_v, snap0, posbuf0)
        emit_positions(i2_v, snap1, posbuf1)
        pltpu.sync_copy(posbuf0, pos0_hbm.at[pl.ds(base_tok, TPW)])
        pltpu.sync_copy(posbuf1, pos1_hbm.at[pl.ds(base_tok, TPW)])

        xbuf = (xba, xbb)
        idx0 = (idx0a, idx0b)
        idx1 = (idx1a, idx1b)
        seml = (sla, slb)
        sem0 = (s0a, s0b)
        sem1 = (s1a, s1b)

        def load(cc, slot):
            return pltpu.async_copy(
                x_hbm.at[pl.ds(base_tok + cc * 16, 16)], xbuf[slot],
                seml[slot])

        def wait_scat(ps, slot):
            if ps[slot] is not None:
                ps[slot][0].wait()
                ps[slot][1].wait()
                ps[slot] = None

        pend_load = load(0, 0)
        pend_scat = [None, None]
        for cc in range(NCH):
            slot = cc % 2
            cur_load = pend_load
            if cc + 1 < NCH:
                wait_scat(pend_scat, 1 - slot)
                pend_load = load(cc + 1, 1 - slot)
            cur_load.wait()
            wait_scat(pend_scat, slot)
            idx0[slot][...] = posbuf0[pl.ds(cc * 16, 16)]
            idx1[slot][...] = posbuf1[pl.ds(cc * 16, 16)]
            cp0 = pltpu.async_copy(xbuf[slot], xd_hbm.at[idx0[slot]],
                                   sem0[slot])
            cp1 = pltpu.async_copy(xbuf[slot], xd_hbm.at[idx1[slot]],
                                   sem1[slot])
            pend_scat[slot] = (cp0, cp1)
        wait_scat(pend_scat, 0)
        wait_scat(pend_scat, 1)

    return dispatch


def _build_sc_combine(T, h, NPAD):
    NW = _NC * _NS
    TPW = T // NW
    NCH = TPW // 16
    mesh = plsc.VectorSubcoreMesh(core_axis_name="c", subcore_axis_name="s")

    @functools.partial(
        pl.kernel, mesh=mesh,
        compiler_params=pltpu.CompilerParams(needs_layout_passes=False),
        out_type=jax.ShapeDtypeStruct((T, h), jnp.float32),
        scratch_types=[
            pltpu.VMEM((2, 16), jnp.int32),
            pltpu.VMEM((2, 16), jnp.int32),
            pltpu.VMEM((2, 16), jnp.float32),
            pltpu.VMEM((2, 16), jnp.float32),
            pltpu.VMEM((16, h), jnp.float32),
            pltpu.VMEM((16, h), jnp.float32),
            pltpu.VMEM((16, h), jnp.float32),
            pltpu.VMEM((16, h), jnp.float32),
            pltpu.VMEM((16, h), jnp.float32),
            pltpu.VMEM((16, h), jnp.float32),
            pltpu.VMEM((16, h), jnp.float32),
            pltpu.SemaphoreType.DMA,
            pltpu.SemaphoreType.DMA,
            pltpu.SemaphoreType.DMA,
            pltpu.SemaphoreType.DMA,
            pltpu.SemaphoreType.DMA,
            pltpu.SemaphoreType.DMA,
        ],
    )
    def combine(y_hbm, sh_hbm, pos0_hbm, pos1_hbm, pw0_hbm, pw1_hbm,
                out_hbm,
                idx0, idx1, p0c, p1c, y0a, y0b, y1a, y1b, sba, sbb, obuf,
                s0a, s0b, s1a, s1b, ssa, ssb):
        cid = lax.axis_index("c")
        sid = lax.axis_index("s")
        wid = sid * _NC + cid
        base_tok = wid * TPW
        lanes = lax.broadcasted_iota(jnp.int32, (16,), 0)

        ybuf0 = (y0a, y0b)
        ybuf1 = (y1a, y1b)
        sbuf = (sba, sbb)
        sem0 = (s0a, s0b)
        sem1 = (s1a, s1b)
        sems = (ssa, ssb)

        def issue(cc, slot):
            base = base_tok + cc * 16
            pltpu.sync_copy(pos0_hbm.at[pl.ds(base, 16)], idx0.at[slot])
            pltpu.sync_copy(pos1_hbm.at[pl.ds(base, 16)], idx1.at[slot])
            pltpu.sync_copy(pw0_hbm.at[pl.ds(base, 16)], p0c.at[slot])
            pltpu.sync_copy(pw1_hbm.at[pl.ds(base, 16)], p1c.at[slot])
            cp0 = pltpu.async_copy(y_hbm.at[idx0.at[slot]], ybuf0[slot],
                                   sem0[slot])
            cp1 = pltpu.async_copy(y_hbm.at[idx1.at[slot]], ybuf1[slot],
                                   sem1[slot])
            cps = pltpu.async_copy(sh_hbm.at[pl.ds(base, 16)], sbuf[slot],
                                   sems[slot])
            return cp0, cp1, cps

        pend = issue(0, 0)
        for cc in range(NCH):
            slot = cc % 2
            cur = pend
            if cc + 1 < NCH:
                pend = issue(cc + 1, (cc + 1) % 2)
            cur[0].wait()
            cur[1].wait()
            cur[2].wait()

            p0v = p0c[slot]
            p1v = p1c[slot]
            yb0 = ybuf0[slot]
            yb1 = ybuf1[slot]
            sb = sbuf[slot]

            for r in range(16):
                w0 = jnp.sum(jnp.where(lanes == r, p0v, 0.0))
                w1 = jnp.sum(jnp.where(lanes == r, p1v, 0.0))

                def body(j, carry):
                    off = j * 64
                    for q in range(4):
                        o = off + q * 16
                        obuf[r, pl.ds(o, 16)] = (
                            w0 * yb0[r, pl.ds(o, 16)]
                            + w1 * yb1[r, pl.ds(o, 16)]
                            + sb[r, pl.ds(o, 16)])
                    return carry

                lax.fori_loop(0, h // 64, body, 0)
            pltpu.sync_copy(obuf, out_hbm.at[pl.ds(base_tok + cc * 16, 16)])

    return combine


def kernel(x, router_w, gate_w, up_w, down_w, shared_gate_w, shared_up_w,
           shared_down_w):
    b, s, h = x.shape
    d = gate_w.shape[2]
    T = b * s
    x_flat = x.reshape(T, h)
    NPAD = T * _K + _E * _R
    NT2 = NPAD // _R
    NT2_PAD = ((NT2 + 15) // 16) * 16

    nT = T // _TT
    idx8, p8 = pl.pallas_call(
        _tc1a_body,
        grid=(nT,),
        in_specs=[
            pl.BlockSpec((h, _E), lambda j: (0, 0)),
            pl.BlockSpec((_TT, h), lambda j: (j, 0)),
        ],
        out_specs=[
            pl.BlockSpec((_TT, _E), lambda j: (j, 0)),
            pl.BlockSpec((_TT, _E), lambda j: (j, 0)),
        ],
        out_shape=[
            jax.ShapeDtypeStruct((T, _E), jnp.int32),
            jax.ShapeDtypeStruct((T, _E), jnp.float32),
        ],
    )(router_w, x_flat)

    i1 = idx8[:, 0]
    i2 = idx8[:, 1]
    p1 = p8[:, 0]
    p2 = p8[:, 1]

    dispatch = _build_sc_dispatch(T, h, NPAD, NT2_PAD)
    x_disp, pos0, pos1, e_of_tile, nv16 = dispatch(x_flat, i1, i2)

    shared_out = pl.pallas_call(
        _tc1b_body,
        grid=(nT,),
        in_specs=[
            pl.BlockSpec((_TT, h), lambda j: (j, 0)),
            pl.BlockSpec((h, d), lambda j: (0, 0)),
            pl.BlockSpec((h, d), lambda j: (0, 0)),
            pl.BlockSpec((d, h), lambda j: (0, 0)),
        ],
        out_specs=pl.BlockSpec((_TT, h), lambda j: (j, 0)),
        out_shape=jax.ShapeDtypeStruct((T, h), jnp.float32),
    )(x_flat, shared_gate_w, shared_up_w, shared_down_w)

    grid_spec = pltpu.PrefetchScalarGridSpec(
        num_scalar_prefetch=2,
        grid=(NT2,),
        in_specs=[
            pl.BlockSpec((_R, h), lambda i, et, nv: (i, 0)),
            pl.BlockSpec((1, h, d), lambda i, et, nv: (et[i], 0, 0)),
            pl.BlockSpec((1, h, d), lambda i, et, nv: (et[i], 0, 0)),
            pl.BlockSpec((1, d, h), lambda i, et, nv: (et[i], 0, 0)),
        ],
        out_specs=pl.BlockSpec((_R, h), lambda i, et, nv: (i, 0)),
    )
    y = pl.pallas_call(
        _tc2_body,
        grid_spec=grid_spec,
        out_shape=jax.ShapeDtypeStruct((NPAD, h), jnp.float32),
    )(e_of_tile, nv16[:1], x_disp, gate_w, up_w, down_w)

    combine = _build_sc_combine(T, h, NPAD)
    out = combine(y, shared_out, pos0, pos1, p1, p2)
    return out.reshape(b, s, h)
